# Initial kernel scaffold; baseline (speedup 1.0000x reference)
import jax
import jax.numpy as jnp
from jax.experimental import pallas as pl


def kernel(x, emb_table, lin_w, lin_b):
    raise NotImplementedError("write your pallas kernel here")


# speedup vs baseline: 86.1054x; 86.1054x over previous
import functools

import jax
import jax.numpy as jnp
from jax import lax
from jax.experimental import pallas as pl
from jax.experimental.pallas import tpu as pltpu
from jax.experimental.pallas import tpu_sc as plsc

B, L = 16384, 200
N = B * L
NC, NS, LANES = 2, 16, 16
NW = NC * NS
PER_W = N // NW
BLK = 12800
NBLK = PER_W // BLK
VOCAB, EMB_DIM = 100, 10
VPAD = 112
EMB_WORDS = VPAD * EMB_DIM


def _sc_body(x_hbm, emb_hbm, wb_hbm, out_hbm,
             emb_v, wb_v, lut_v, x_v, out_v):
  wid = lax.axis_index("s") * NC + lax.axis_index("c")
  base = wid * PER_W

  pltpu.sync_copy(emb_hbm, emb_v)
  pltpu.sync_copy(wb_hbm, wb_v)

  iota = lax.iota(jnp.int32, LANES)
  bias = wb_v[pl.ds(EMB_DIM * LANES, LANES)]
  wbc = [wb_v[pl.ds(d * LANES, LANES)] for d in range(EMB_DIM)]
  for c in range(VPAD // LANES):
    row = (c * LANES + iota) * EMB_DIM
    acc = bias
    for d in range(EMB_DIM):
      acc = acc + plsc.load_gather(emb_v, [row + d]) * wbc[d]
    lut_v[pl.ds(c * LANES, LANES)] = acc

  for blk in range(NBLK):
    off = base + blk * BLK
    pltpu.sync_copy(x_hbm.at[pl.ds(off, BLK)], x_v)

    def _gather_one(j, carry):
      idx = x_v[pl.ds(j * LANES, LANES)]
      out_v[pl.ds(j * LANES, LANES)] = plsc.load_gather(lut_v, [idx])
      return carry
    lax.fori_loop(0, BLK // LANES, _gather_one, 0)

    pltpu.sync_copy(out_v, out_hbm.at[pl.ds(off, BLK)])


@jax.jit
def _lut_gather(x_flat, emb_flat, wb):
  mesh = plsc.VectorSubcoreMesh(core_axis_name="c", subcore_axis_name="s",
                                num_cores=NC, num_subcores=NS)
  return pl.kernel(
      _sc_body,
      out_type=jax.ShapeDtypeStruct((N,), jnp.float32),
      mesh=mesh,
      compiler_params=pltpu.CompilerParams(needs_layout_passes=False),
      scratch_types=[
          pltpu.VMEM((EMB_WORDS,), jnp.float32),
          pltpu.VMEM(((EMB_DIM + 1) * LANES,), jnp.float32),
          pltpu.VMEM((VPAD,), jnp.float32),
          pltpu.VMEM((BLK,), jnp.int32),
          pltpu.VMEM((BLK,), jnp.float32),
      ],
  )(x_flat, emb_flat, wb)


def kernel(x, emb_table, lin_w, lin_b):
  x_flat = x.reshape(-1).astype(jnp.int32)
  emb_flat = jnp.pad(emb_table.reshape(-1), (0, EMB_WORDS - VOCAB * EMB_DIM))
  wb = jnp.repeat(jnp.concatenate([lin_w.reshape(-1), lin_b.reshape(-1)]),
                  LANES)
  out = _lut_gather(x_flat, emb_flat, wb)
  return out.reshape(B, L, 1)


# speedup vs baseline: 107.7640x; 1.2515x over previous
import functools

import jax
import jax.numpy as jnp
from jax import lax
from jax.experimental import pallas as pl
from jax.experimental.pallas import tpu as pltpu
from jax.experimental.pallas import tpu_sc as plsc

B, L = 16384, 200
N = B * L
NC, NS, LANES = 2, 16, 16
NW = NC * NS
PER_W = N // NW
BLK = 12800
NBLK = PER_W // BLK
VOCAB, EMB_DIM = 100, 10
VPAD = 112
EMB_WORDS = VPAD * EMB_DIM


def _sc_body(x_hbm, emb_hbm, wb_hbm, out_hbm,
             emb_v, wb_v, lut_v, x_v, out_v):
  wid = lax.axis_index("s") * NC + lax.axis_index("c")
  base = wid * PER_W

  pltpu.sync_copy(emb_hbm, emb_v)
  pltpu.sync_copy(wb_hbm, wb_v)

  iota = lax.iota(jnp.int32, LANES)
  bias = wb_v[pl.ds(EMB_DIM * LANES, LANES)]
  wbc = [wb_v[pl.ds(d * LANES, LANES)] for d in range(EMB_DIM)]
  for c in range(VPAD // LANES):
    row = (c * LANES + iota) * EMB_DIM
    acc = bias
    for d in range(EMB_DIM):
      acc = acc + plsc.load_gather(emb_v, [row + d]) * wbc[d]
    lut_v[pl.ds(c * LANES, LANES)] = acc

  for blk in range(NBLK):
    off = base + blk * BLK
    pltpu.sync_copy(x_hbm.at[pl.ds(off, BLK)], x_v)

    @plsc.parallel_loop(0, BLK // LANES, 1, unroll=8)
    def _(j):
      idx = x_v[pl.ds(j * LANES, LANES)]
      out_v[pl.ds(j * LANES, LANES)] = plsc.load_gather(lut_v, [idx])

    pltpu.sync_copy(out_v, out_hbm.at[pl.ds(off, BLK)])


@jax.jit
def _lut_gather(x_flat, emb_flat, wb):
  mesh = plsc.VectorSubcoreMesh(core_axis_name="c", subcore_axis_name="s",
                                num_cores=NC, num_subcores=NS)
  return pl.kernel(
      _sc_body,
      out_type=jax.ShapeDtypeStruct((N,), jnp.float32),
      mesh=mesh,
      compiler_params=pltpu.CompilerParams(needs_layout_passes=False),
      scratch_types=[
          pltpu.VMEM((EMB_WORDS,), jnp.float32),
          pltpu.VMEM(((EMB_DIM + 1) * LANES,), jnp.float32),
          pltpu.VMEM((VPAD,), jnp.float32),
          pltpu.VMEM((BLK,), jnp.int32),
          pltpu.VMEM((BLK,), jnp.float32),
      ],
  )(x_flat, emb_flat, wb)


def kernel(x, emb_table, lin_w, lin_b):
  x_flat = x.reshape(-1).astype(jnp.int32)
  emb_flat = jnp.pad(emb_table.reshape(-1), (0, EMB_WORDS - VOCAB * EMB_DIM))
  wb = jnp.repeat(jnp.concatenate([lin_w.reshape(-1), lin_b.reshape(-1)]),
                  LANES)
  out = _lut_gather(x_flat, emb_flat, wb)
  return out.reshape(B, L, 1)


# speedup vs baseline: 118.1229x; 1.0961x over previous
import functools

import jax
import jax.numpy as jnp
from jax import lax
from jax.experimental import pallas as pl
from jax.experimental.pallas import tpu as pltpu
from jax.experimental.pallas import tpu_sc as plsc

B, L = 16384, 200
N = B * L
NC, NS, LANES = 2, 16, 16
NW = NC * NS
PER_W = N // NW
BLK = 12800
NBLK = PER_W // BLK
VOCAB, EMB_DIM = 100, 10
VPAD = 112
EMB_WORDS = VPAD * EMB_DIM


def _sc_body(x_hbm, emb_hbm, wb_hbm, out_hbm,
             emb_v, wb_v, lut_v, x_v0, x_v1, o_v0, o_v1,
             sx0, sx1, so0, so1):
  wid = lax.axis_index("s") * NC + lax.axis_index("c")
  base = wid * PER_W
  xbuf, obuf = [x_v0, x_v1], [o_v0, o_v1]
  xsem, osem = [sx0, sx1], [so0, so1]

  def start_x(blk):
    return pltpu.async_copy(
        x_hbm.at[pl.ds(base + blk * BLK, BLK)], xbuf[blk % 2], xsem[blk % 2])

  xcopy = {0: start_x(0), 1: start_x(1)}
  ocopy = {}

  pltpu.sync_copy(emb_hbm, emb_v)
  pltpu.sync_copy(wb_hbm, wb_v)

  iota = lax.iota(jnp.int32, LANES)
  bias = wb_v[pl.ds(EMB_DIM * LANES, LANES)]
  wbc = [wb_v[pl.ds(d * LANES, LANES)] for d in range(EMB_DIM)]
  for c in range(VPAD // LANES):
    row = (c * LANES + iota) * EMB_DIM
    acc = bias
    for d in range(EMB_DIM):
      acc = acc + plsc.load_gather(emb_v, [row + d]) * wbc[d]
    lut_v[pl.ds(c * LANES, LANES)] = acc

  for blk in range(NBLK):
    xcopy[blk].wait()
    if blk >= 2:
      ocopy[blk - 2].wait()
    xv, ov = xbuf[blk % 2], obuf[blk % 2]

    @plsc.parallel_loop(0, BLK // LANES, 1, unroll=8)
    def _(j):
      idx = xv[pl.ds(j * LANES, LANES)]
      ov[pl.ds(j * LANES, LANES)] = plsc.load_gather(lut_v, [idx])

    if blk + 2 < NBLK:
      xcopy[blk + 2] = start_x(blk + 2)
    ocopy[blk] = pltpu.async_copy(
        ov, out_hbm.at[pl.ds(base + blk * BLK, BLK)], osem[blk % 2])

  ocopy[NBLK - 2].wait()
  ocopy[NBLK - 1].wait()


@jax.jit
def _lut_gather(x_flat, emb_flat, wb):
  mesh = plsc.VectorSubcoreMesh(core_axis_name="c", subcore_axis_name="s",
                                num_cores=NC, num_subcores=NS)
  return pl.kernel(
      _sc_body,
      out_type=jax.ShapeDtypeStruct((N,), jnp.float32),
      mesh=mesh,
      compiler_params=pltpu.CompilerParams(needs_layout_passes=False),
      scratch_types=[
          pltpu.VMEM((EMB_WORDS,), jnp.float32),
          pltpu.VMEM(((EMB_DIM + 1) * LANES,), jnp.float32),
          pltpu.VMEM((VPAD,), jnp.float32),
          pltpu.VMEM((BLK,), jnp.int32),
          pltpu.VMEM((BLK,), jnp.int32),
          pltpu.VMEM((BLK,), jnp.float32),
          pltpu.VMEM((BLK,), jnp.float32),
          pltpu.SemaphoreType.DMA,
          pltpu.SemaphoreType.DMA,
          pltpu.SemaphoreType.DMA,
          pltpu.SemaphoreType.DMA,
      ],
  )(x_flat, emb_flat, wb)


def kernel(x, emb_table, lin_w, lin_b):
  x_flat = x.reshape(-1).astype(jnp.int32)
  emb_flat = jnp.pad(emb_table.reshape(-1), (0, EMB_WORDS - VOCAB * EMB_DIM))
  wb = jnp.repeat(jnp.concatenate([lin_w.reshape(-1), lin_b.reshape(-1)]),
                  LANES)
  out = _lut_gather(x_flat, emb_flat, wb)
  return out.reshape(B, L, 1)


# speedup vs baseline: 189.6755x; 1.6057x over previous
import jax
import jax.numpy as jnp
from jax import lax
from jax.experimental import pallas as pl
from jax.experimental.pallas import tpu as pltpu
from jax.experimental.pallas import tpu_sc as plsc

B, L = 16384, 200
NC, NS, LANES = 2, 16, 16
NW = NC * NS
ROWS_W = B // NW
ROWS_BLK = 64
NBLK = ROWS_W // ROWS_BLK
NCHUNK = 12
CTAIL = NCHUNK * LANES
VOCAB, EMB_DIM = 100, 10
VPAD = 112
EMB_WORDS = VPAD * EMB_DIM


def _sc_body(x_hbm, emb_hbm, wb_hbm, out_hbm,
             emb_v, wb_v, lut_v, x_v0, x_v1, o_v0, o_v1,
             sx0, sx1, so0, so1):
  wid = lax.axis_index("s") * NC + lax.axis_index("c")
  row0 = wid * ROWS_W
  xbuf, obuf = [x_v0, x_v1], [o_v0, o_v1]
  xsem, osem = [sx0, sx1], [so0, so1]

  def start_x(blk):
    return pltpu.async_copy(
        x_hbm.at[pl.ds(row0 + blk * ROWS_BLK, ROWS_BLK)],
        xbuf[blk % 2], xsem[blk % 2])

  xcopy = {0: start_x(0), 1: start_x(1)}
  ocopy = {}

  pltpu.sync_copy(emb_hbm, emb_v)
  pltpu.sync_copy(wb_hbm, wb_v)

  iota = lax.iota(jnp.int32, LANES)
  bias = wb_v[pl.ds(EMB_DIM * LANES, LANES)]
  wbc = [wb_v[pl.ds(d * LANES, LANES)] for d in range(EMB_DIM)]
  for c in range(VPAD // LANES):
    row = (c * LANES + iota) * EMB_DIM
    acc = bias
    for d in range(EMB_DIM):
      acc = acc + plsc.load_gather(emb_v, [row + d]) * wbc[d]
    lut_v[pl.ds(c * LANES, LANES)] = acc

  tail_r = iota // 8
  tail_c = CTAIL + (iota % 8)

  for blk in range(NBLK):
    xcopy[blk].wait()
    if blk >= 2:
      ocopy[blk - 2].wait()
    xv, ov = xbuf[blk % 2], obuf[blk % 2]

    @plsc.parallel_loop(0, ROWS_BLK, 1, unroll=2)
    def _(r):
      for cc in range(NCHUNK):
        idx = xv[r, pl.ds(cc * LANES, LANES)]
        ov[r, pl.ds(cc * LANES, LANES)] = plsc.load_gather(lut_v, [idx])

    @plsc.parallel_loop(0, ROWS_BLK // 2, 1, unroll=4)
    def _(t):
      rv = 2 * t + tail_r
      idx = plsc.load_gather(xv, [rv, tail_c])
      plsc.store_scatter(ov, [rv, tail_c], plsc.load_gather(lut_v, [idx]))

    if blk + 2 < NBLK:
      xcopy[blk + 2] = start_x(blk + 2)
    ocopy[blk] = pltpu.async_copy(
        ov, out_hbm.at[pl.ds(row0 + blk * ROWS_BLK, ROWS_BLK)],
        osem[blk % 2])

  ocopy[NBLK - 2].wait()
  ocopy[NBLK - 1].wait()


@jax.jit
def _lut_gather(x, emb_flat, wb):
  mesh = plsc.VectorSubcoreMesh(core_axis_name="c", subcore_axis_name="s",
                                num_cores=NC, num_subcores=NS)
  return pl.kernel(
      _sc_body,
      out_type=jax.ShapeDtypeStruct((B, L), jnp.float32),
      mesh=mesh,
      compiler_params=pltpu.CompilerParams(needs_layout_passes=False),
      scratch_types=[
          pltpu.VMEM((EMB_WORDS,), jnp.float32),
          pltpu.VMEM(((EMB_DIM + 1) * LANES,), jnp.float32),
          pltpu.VMEM((VPAD,), jnp.float32),
          pltpu.VMEM((ROWS_BLK, L), jnp.int32),
          pltpu.VMEM((ROWS_BLK, L), jnp.int32),
          pltpu.VMEM((ROWS_BLK, L), jnp.float32),
          pltpu.VMEM((ROWS_BLK, L), jnp.float32),
          pltpu.SemaphoreType.DMA,
          pltpu.SemaphoreType.DMA,
          pltpu.SemaphoreType.DMA,
          pltpu.SemaphoreType.DMA,
      ],
  )(x, emb_flat, wb)


def kernel(x, emb_table, lin_w, lin_b):
  emb_flat = jnp.pad(emb_table.reshape(-1), (0, EMB_WORDS - VOCAB * EMB_DIM))
  wb = jnp.repeat(jnp.concatenate([lin_w.reshape(-1), lin_b.reshape(-1)]),
                  LANES)
  out = _lut_gather(x.astype(jnp.int32), emb_flat, wb)
  return out[:, :, None]
